# baseline (speedup 1.0000x reference)
import functools

import jax
import jax.numpy as jnp
from jax import lax
from jax.experimental import pallas as pl
from jax.experimental.pallas import tpu as pltpu
from jax.experimental.pallas import tpu_sc as plsc

T = 2048
H = 768
I = 3072
E = 8
K = 2
BLK = 128
NB = (T * K) // BLK + E
PAD = NB * BLK
CH = 128

NC = 2
NS = 16
NW = NC * NS
APW = (T * K) // NW
TPW = T // NW
TCH = 16
NCHUNK = TPW // TCH


def _router_body(x_ref, wr_ref, pos12_ref, wexp_ref, be_ref, na_ref,
                 usage_ref, lbl_ref):
    x = x_ref[...]
    wr = wr_ref[...]
    logits = lax.dot_general(x, wr, (((1,), (1,)), ((), ())))

    iota8 = lax.broadcasted_iota(jnp.int32, (T, E), 1).astype(jnp.float32)
    v1 = jnp.max(logits, axis=1, keepdims=True)
    i1 = jnp.min(jnp.where(logits == v1, iota8, float(E)), axis=1,
                 keepdims=True)
    oh1 = (iota8 == i1).astype(jnp.float32)
    masked = jnp.where(iota8 == i1, -jnp.inf, logits)
    v2 = jnp.max(masked, axis=1, keepdims=True)
    i2 = jnp.min(jnp.where(masked == v2, iota8, float(E)), axis=1,
                 keepdims=True)
    oh2 = (iota8 == i2).astype(jnp.float32)

    u = jnp.exp(v2 - v1)
    s = 1.0 + u
    w1 = 1.0 / s
    w2 = u / s
    ones16 = jnp.ones((1, 16), jnp.float32)
    wexp_ref[...] = jnp.concatenate([w1 * ones16, w2 * ones16], axis=1)

    both = oh1 + oh2
    cnt = jnp.sum(both, axis=0, keepdims=True)

    ri = lax.broadcasted_iota(jnp.int32, (CH, CH), 0)
    ci = lax.broadcasted_iota(jnp.int32, (CH, CH), 1)
    tri = (ci < ri).astype(jnp.float32)
    parts = []
    off = jnp.zeros((1, E), jnp.float32)
    for c in range(T // CH):
        blk = lax.slice(both, (c * CH, 0), ((c + 1) * CH, E))
        s_blk = lax.dot_general(tri, blk, (((1,), (0,)), ((), ()))) + off
        parts.append(s_blk)
        off = off + jnp.sum(blk, axis=0, keepdims=True)
    s_all = jnp.concatenate(parts, axis=0)

    nblk = jnp.floor((cnt + float(BLK - 1)) * (1.0 / BLK))
    re8 = lax.broadcasted_iota(jnp.int32, (E, E), 0)
    ce8 = lax.broadcasted_iota(jnp.int32, (E, E), 1)
    triu8 = (re8 < ce8).astype(jnp.float32)
    offs = lax.dot_general(nblk, triu8, (((1,), (0,)), ((), ()))) * float(BLK)

    rank1 = jnp.sum(s_all * oh1, axis=1, keepdims=True)
    rank2 = jnp.sum(s_all * oh2, axis=1, keepdims=True)
    base1 = jnp.sum(offs * oh1, axis=1, keepdims=True)
    base2 = jnp.sum(offs * oh2, axis=1, keepdims=True)
    pos12_ref[...] = jnp.concatenate(
        [base1 + rank1, base2 + rank2], axis=1).astype(jnp.int32)

    total = jnp.sum(nblk, axis=1, keepdims=True)
    bar = lax.broadcasted_iota(jnp.int32, (NB, 1), 0).astype(jnp.float32)
    barc = jnp.minimum(bar, total - 1.0)
    cum_excl = offs * (1.0 / BLK)
    be = jnp.sum((barc >= cum_excl).astype(jnp.float32), axis=1,
                 keepdims=True) - 1.0
    be_ref[...] = be.astype(jnp.int32)
    na_ref[...] = total.astype(jnp.int32)

    p = jnp.exp(logits - v1)
    p = p / jnp.sum(p, axis=1, keepdims=True)
    usage = jnp.mean(p, axis=0, keepdims=True)
    usage_ref[...] = usage
    d = usage - (1.0 / E)
    lbl_ref[...] = jnp.sum(d * d, keepdims=True)


def _router_call(x_flat, W_router):
    return pl.pallas_call(
        _router_body,
        out_shape=[
            jax.ShapeDtypeStruct((T, K), jnp.int32),
            jax.ShapeDtypeStruct((T, K * 16), jnp.float32),
            jax.ShapeDtypeStruct((NB, 1), jnp.int32),
            jax.ShapeDtypeStruct((1, 1), jnp.int32),
            jax.ShapeDtypeStruct((1, E), jnp.float32),
            jax.ShapeDtypeStruct((1, 1), jnp.float32),
        ],
    )(x_flat, W_router)


def _gelu_exact(h):
    return 0.5 * h * (1.0 + lax.erf(h * 0.7071067811865476))


def _ffn_body(be_ref, na_ref, x_ref, wu_ref, wd_ref, y_ref):
    b = pl.program_id(0)

    @pl.when(b < na_ref[0])
    def _():
        xb = x_ref[...]
        wu = wu_ref[0]
        h = lax.dot_general(xb, wu, (((1,), (1,)), ((), ())))
        h = _gelu_exact(h)
        wd = wd_ref[0]
        y_ref[...] = lax.dot_general(h, wd, (((1,), (1,)), ((), ())))


def _ffn_call(block_expert, n_active, x_sorted, W_up, W_down):
    grid_spec = pltpu.PrefetchScalarGridSpec(
        num_scalar_prefetch=2,
        grid=(NB,),
        in_specs=[
            pl.BlockSpec((BLK, H), lambda b, be, na: (b, 0)),
            pl.BlockSpec((1, I, H), lambda b, be, na: (be[b], 0, 0)),
            pl.BlockSpec((1, H, I), lambda b, be, na: (be[b], 0, 0)),
        ],
        out_specs=pl.BlockSpec((BLK, H), lambda b, be, na: (b, 0)),
    )
    return pl.pallas_call(
        _ffn_body,
        grid_spec=grid_spec,
        out_shape=jax.ShapeDtypeStruct((PAD, H), jnp.float32),
        compiler_params=pltpu.CompilerParams(
            dimension_semantics=("arbitrary",)),
    )(block_expert, n_active, x_sorted, W_up, W_down)


def _make_dispatch():
    mesh = plsc.VectorSubcoreMesh(core_axis_name="c", subcore_axis_name="s")

    @functools.partial(
        pl.kernel,
        mesh=mesh,
        out_type=jax.ShapeDtypeStruct((PAD, H), jnp.float32),
        scratch_types=[
            pltpu.VMEM((APW,), jnp.int32),
            pltpu.VMEM((APW,), jnp.int32),
            pltpu.VMEM((APW, H), jnp.float32),
            pltpu.SemaphoreType.DMA,
            pltpu.SemaphoreType.DMA,
        ],
    )
    def dispatch(x_hbm, tok_hbm, pos_hbm, out_hbm, tok_v, pos_v, rows_v,
                 sem1, sem2):
        wid = lax.axis_index("s") * NC + lax.axis_index("c")
        base = wid * APW
        pltpu.sync_copy(tok_hbm.at[pl.ds(base, APW)], tok_v)
        pltpu.async_copy(x_hbm.at[tok_v], rows_v, sem1).wait()
        pltpu.sync_copy(pos_hbm.at[pl.ds(base, APW)], pos_v)
        pltpu.async_copy(rows_v, out_hbm.at[pos_v], sem2).wait()

    return dispatch


def _make_combine():
    mesh = plsc.VectorSubcoreMesh(core_axis_name="c", subcore_axis_name="s")

    @functools.partial(
        pl.kernel,
        mesh=mesh,
        out_type=jax.ShapeDtypeStruct((T, H), jnp.float32),
        scratch_types=[
            pltpu.VMEM((K * TCH,), jnp.int32),
            pltpu.VMEM((K * TCH,), jnp.int32),
            pltpu.VMEM((K * TCH, 16), jnp.float32),
            pltpu.VMEM((K * TCH, H), jnp.float32),
            pltpu.VMEM((K * TCH, H), jnp.float32),
            pltpu.VMEM((TCH, H), jnp.float32),
            pltpu.SemaphoreType.DMA,
            pltpu.SemaphoreType.DMA,
        ],
    )
    def combine(y_hbm, pos_hbm, wexp_hbm, out_hbm, pos0_v, pos1_v, w_v,
                rows0_v, rows1_v, acc_v, sem0, sem1):
        wid = lax.axis_index("s") * NC + lax.axis_index("c")
        t0 = wid * TPW
        pos_bufs = (pos0_v, pos1_v)
        row_bufs = (rows0_v, rows1_v)
        sems = (sem0, sem1)
        pltpu.sync_copy(pos_hbm.at[pl.ds(K * t0, K * TCH)], pos0_v)
        cps = [pltpu.async_copy(y_hbm.at[pos0_v], rows0_v, sem0), None]
        for chunk in range(NCHUNK):
            cur = chunk % 2
            nxt = (chunk + 1) % 2
            tc = t0 + chunk * TCH
            if chunk + 1 < NCHUNK:
                pltpu.sync_copy(
                    pos_hbm.at[pl.ds(K * (tc + TCH), K * TCH)],
                    pos_bufs[nxt])
                cps[nxt] = pltpu.async_copy(
                    y_hbm.at[pos_bufs[nxt]], row_bufs[nxt], sems[nxt])
            pltpu.sync_copy(wexp_hbm.at[pl.ds(K * tc, K * TCH)], w_v)
            cps[cur].wait()
            rows_v = row_bufs[cur]

            @plsc.parallel_loop(0, TCH, 1, unroll=2)
            def tok_body(i, rows_v=rows_v):
                w0 = w_v[2 * i]
                w1 = w_v[2 * i + 1]
                for c in range(H // 16):
                    r0 = rows_v[2 * i, pl.ds(c * 16, 16)]
                    r1 = rows_v[2 * i + 1, pl.ds(c * 16, 16)]
                    acc_v[i, pl.ds(c * 16, 16)] = r0 * w0 + r1 * w1
            pltpu.sync_copy(acc_v, out_hbm.at[pl.ds(tc, TCH)])

    return combine


_dispatch_call = _make_dispatch()
_combine_call = _make_combine()


def kernel(x, W_router, W_up, W_down):
    xf = x.reshape(T, H)
    pos12, wexp, be, na, usage, lbl = _router_call(xf, W_router)

    positions = pos12.reshape(T * K)
    w_expand = wexp.reshape(T * K, 16)
    block_expert = be.reshape(NB)
    n_active = na.reshape(1)
    tok_ids = jnp.arange(T * K, dtype=jnp.int32) // K

    x_sorted = _dispatch_call(xf, tok_ids, positions)
    y_sorted = _ffn_call(block_expert, n_active, x_sorted, W_up, W_down)
    out = _combine_call(y_sorted, positions, w_expand)

    return out.reshape(1, T, H), lbl.reshape(()), usage.reshape(E)


# speedup vs baseline: 1.4343x; 1.4343x over previous
import functools

import jax
import jax.numpy as jnp
from jax import lax
from jax.experimental import pallas as pl
from jax.experimental.pallas import tpu as pltpu
from jax.experimental.pallas import tpu_sc as plsc

T = 2048
H = 768
I = 3072
E = 8
K = 2
BLK = 512
NB = (T * K) // BLK + E
PAD = NB * BLK
CH = 128

NC = 2
NS = 16
NW = NC * NS
APW = (T * K) // NW
TPW = T // NW
TCH = 16
NCHUNK = TPW // TCH


def _router_body(x_ref, wr_ref, pos12_ref, wexp_ref, be_ref, na_ref,
                 usage_ref, lbl_ref):
    x = x_ref[...]
    wr = wr_ref[...]
    logits = lax.dot_general(x, wr, (((1,), (1,)), ((), ())))

    iota8 = lax.broadcasted_iota(jnp.int32, (T, E), 1).astype(jnp.float32)
    v1 = jnp.max(logits, axis=1, keepdims=True)
    i1 = jnp.min(jnp.where(logits == v1, iota8, float(E)), axis=1,
                 keepdims=True)
    oh1 = (iota8 == i1).astype(jnp.float32)
    masked = jnp.where(iota8 == i1, -jnp.inf, logits)
    v2 = jnp.max(masked, axis=1, keepdims=True)
    i2 = jnp.min(jnp.where(masked == v2, iota8, float(E)), axis=1,
                 keepdims=True)
    oh2 = (iota8 == i2).astype(jnp.float32)

    u = jnp.exp(v2 - v1)
    s = 1.0 + u
    w1 = 1.0 / s
    w2 = u / s
    ones16 = jnp.ones((1, 16), jnp.float32)
    wexp_ref[...] = jnp.concatenate([w1 * ones16, w2 * ones16], axis=1)

    both = oh1 + oh2
    cnt = jnp.sum(both, axis=0, keepdims=True)

    ri = lax.broadcasted_iota(jnp.int32, (CH, CH), 0)
    ci = lax.broadcasted_iota(jnp.int32, (CH, CH), 1)
    tri = (ci < ri).astype(jnp.float32)
    parts = []
    off = jnp.zeros((1, E), jnp.float32)
    for c in range(T // CH):
        blk = lax.slice(both, (c * CH, 0), ((c + 1) * CH, E))
        s_blk = lax.dot_general(tri, blk, (((1,), (0,)), ((), ()))) + off
        parts.append(s_blk)
        off = off + jnp.sum(blk, axis=0, keepdims=True)
    s_all = jnp.concatenate(parts, axis=0)

    nblk = jnp.floor((cnt + float(BLK - 1)) * (1.0 / BLK))
    re8 = lax.broadcasted_iota(jnp.int32, (E, E), 0)
    ce8 = lax.broadcasted_iota(jnp.int32, (E, E), 1)
    triu8 = (re8 < ce8).astype(jnp.float32)
    offs = lax.dot_general(nblk, triu8, (((1,), (0,)), ((), ()))) * float(BLK)

    rank1 = jnp.sum(s_all * oh1, axis=1, keepdims=True)
    rank2 = jnp.sum(s_all * oh2, axis=1, keepdims=True)
    base1 = jnp.sum(offs * oh1, axis=1, keepdims=True)
    base2 = jnp.sum(offs * oh2, axis=1, keepdims=True)
    pos12_ref[...] = jnp.concatenate(
        [base1 + rank1, base2 + rank2], axis=1).astype(jnp.int32)

    total = jnp.sum(nblk, axis=1, keepdims=True)
    bar = lax.broadcasted_iota(jnp.int32, (NB, 1), 0).astype(jnp.float32)
    barc = jnp.minimum(bar, total - 1.0)
    cum_excl = offs * (1.0 / BLK)
    be = jnp.sum((barc >= cum_excl).astype(jnp.float32), axis=1,
                 keepdims=True) - 1.0
    be_ref[...] = be.astype(jnp.int32)
    na_ref[...] = total.astype(jnp.int32)

    p = jnp.exp(logits - v1)
    p = p / jnp.sum(p, axis=1, keepdims=True)
    usage = jnp.mean(p, axis=0, keepdims=True)
    usage_ref[...] = usage
    d = usage - (1.0 / E)
    lbl_ref[...] = jnp.sum(d * d, keepdims=True)


def _router_call(x_flat, W_router):
    return pl.pallas_call(
        _router_body,
        out_shape=[
            jax.ShapeDtypeStruct((T, K), jnp.int32),
            jax.ShapeDtypeStruct((T, K * 16), jnp.float32),
            jax.ShapeDtypeStruct((NB, 1), jnp.int32),
            jax.ShapeDtypeStruct((1, 1), jnp.int32),
            jax.ShapeDtypeStruct((1, E), jnp.float32),
            jax.ShapeDtypeStruct((1, 1), jnp.float32),
        ],
    )(x_flat, W_router)


def _gelu_exact(h):
    return 0.5 * h * (1.0 + lax.erf(h * 0.7071067811865476))


def _ffn_body(be_ref, na_ref, x_ref, wu_ref, wd_ref, y_ref):
    b = pl.program_id(0)

    @pl.when(b < na_ref[0])
    def _():
        xb = x_ref[...]
        wu = wu_ref[0]
        h = lax.dot_general(xb, wu, (((1,), (1,)), ((), ())))
        h = _gelu_exact(h)
        wd = wd_ref[0]
        y_ref[...] = lax.dot_general(h, wd, (((1,), (1,)), ((), ())))


def _ffn_call(block_expert, n_active, x_sorted, W_up, W_down):
    grid_spec = pltpu.PrefetchScalarGridSpec(
        num_scalar_prefetch=2,
        grid=(NB,),
        in_specs=[
            pl.BlockSpec((BLK, H), lambda b, be, na: (b, 0)),
            pl.BlockSpec((1, I, H), lambda b, be, na: (be[b], 0, 0)),
            pl.BlockSpec((1, H, I), lambda b, be, na: (be[b], 0, 0)),
        ],
        out_specs=pl.BlockSpec((BLK, H), lambda b, be, na: (b, 0)),
    )
    return pl.pallas_call(
        _ffn_body,
        grid_spec=grid_spec,
        out_shape=jax.ShapeDtypeStruct((PAD, H), jnp.float32),
        compiler_params=pltpu.CompilerParams(
            dimension_semantics=("arbitrary",)),
    )(block_expert, n_active, x_sorted, W_up, W_down)


def _make_dispatch():
    mesh = plsc.VectorSubcoreMesh(core_axis_name="c", subcore_axis_name="s")

    @functools.partial(
        pl.kernel,
        mesh=mesh,
        out_type=jax.ShapeDtypeStruct((PAD, H), jnp.float32),
        scratch_types=[
            pltpu.VMEM((APW,), jnp.int32),
            pltpu.VMEM((APW,), jnp.int32),
            pltpu.VMEM((APW, H), jnp.float32),
            pltpu.SemaphoreType.DMA,
            pltpu.SemaphoreType.DMA,
        ],
    )
    def dispatch(x_hbm, tok_hbm, pos_hbm, out_hbm, tok_v, pos_v, rows_v,
                 sem1, sem2):
        wid = lax.axis_index("s") * NC + lax.axis_index("c")
        base = wid * APW
        pltpu.sync_copy(tok_hbm.at[pl.ds(base, APW)], tok_v)
        pltpu.async_copy(x_hbm.at[tok_v], rows_v, sem1).wait()
        pltpu.sync_copy(pos_hbm.at[pl.ds(base, APW)], pos_v)
        pltpu.async_copy(rows_v, out_hbm.at[pos_v], sem2).wait()

    return dispatch


def _make_combine():
    mesh = plsc.VectorSubcoreMesh(core_axis_name="c", subcore_axis_name="s")

    @functools.partial(
        pl.kernel,
        mesh=mesh,
        out_type=jax.ShapeDtypeStruct((T, H), jnp.float32),
        scratch_types=[
            pltpu.VMEM((K * TCH,), jnp.int32),
            pltpu.VMEM((K * TCH,), jnp.int32),
            pltpu.VMEM((K * TCH, 16), jnp.float32),
            pltpu.VMEM((K * TCH, H), jnp.float32),
            pltpu.VMEM((K * TCH, H), jnp.float32),
            pltpu.VMEM((TCH, H), jnp.float32),
            pltpu.SemaphoreType.DMA,
            pltpu.SemaphoreType.DMA,
        ],
    )
    def combine(y_hbm, pos_hbm, wexp_hbm, out_hbm, pos0_v, pos1_v, w_v,
                rows0_v, rows1_v, acc_v, sem0, sem1):
        wid = lax.axis_index("s") * NC + lax.axis_index("c")
        t0 = wid * TPW
        pos_bufs = (pos0_v, pos1_v)
        row_bufs = (rows0_v, rows1_v)
        sems = (sem0, sem1)
        pltpu.sync_copy(pos_hbm.at[pl.ds(K * t0, K * TCH)], pos0_v)
        cps = [pltpu.async_copy(y_hbm.at[pos0_v], rows0_v, sem0), None]
        for chunk in range(NCHUNK):
            cur = chunk % 2
            nxt = (chunk + 1) % 2
            tc = t0 + chunk * TCH
            if chunk + 1 < NCHUNK:
                pltpu.sync_copy(
                    pos_hbm.at[pl.ds(K * (tc + TCH), K * TCH)],
                    pos_bufs[nxt])
                cps[nxt] = pltpu.async_copy(
                    y_hbm.at[pos_bufs[nxt]], row_bufs[nxt], sems[nxt])
            pltpu.sync_copy(wexp_hbm.at[pl.ds(K * tc, K * TCH)], w_v)
            cps[cur].wait()
            rows_v = row_bufs[cur]

            @plsc.parallel_loop(0, TCH, 1, unroll=2)
            def tok_body(i, rows_v=rows_v):
                w0 = w_v[2 * i]
                w1 = w_v[2 * i + 1]
                for c in range(H // 16):
                    r0 = rows_v[2 * i, pl.ds(c * 16, 16)]
                    r1 = rows_v[2 * i + 1, pl.ds(c * 16, 16)]
                    acc_v[i, pl.ds(c * 16, 16)] = r0 * w0 + r1 * w1
            pltpu.sync_copy(acc_v, out_hbm.at[pl.ds(tc, TCH)])

    return combine


_dispatch_call = _make_dispatch()
_combine_call = _make_combine()


def kernel(x, W_router, W_up, W_down):
    xf = x.reshape(T, H)
    pos12, wexp, be, na, usage, lbl = _router_call(xf, W_router)

    positions = pos12.reshape(T * K)
    w_expand = wexp.reshape(T * K, 16)
    block_expert = be.reshape(NB)
    n_active = na.reshape(1)
    tok_ids = jnp.arange(T * K, dtype=jnp.int32) // K

    x_sorted = _dispatch_call(xf, tok_ids, positions)
    y_sorted = _ffn_call(block_expert, n_active, x_sorted, W_up, W_down)
    out = _combine_call(y_sorted, positions, w_expand)

    return out.reshape(1, T, H), lbl.reshape(()), usage.reshape(E)
